# baseline (speedup 1.0000x reference)
import functools

import jax
import jax.numpy as jnp
from jax import lax
from jax.experimental import pallas as pl
from jax.experimental.pallas import tpu as pltpu
from jax.experimental.pallas import tpu_sc as plsc

KC = 8192
DC = 256
NT = 16384
COMMITMENT_COST = 0.25

TN = 512
TK = 2048
NI = NT // TN
NJ = KC // TK

WB = 2736
_BIG = 0x7FFFFFFF

NW = 32
BPW = NT // NW
CH = 128
NCH = BPW // CH


CL = 128


def _upd(d, lane, j, lo, hi, best_ref, barg_ref, w):
    tn, tk = d.shape
    lo_ = 0 if lo is None else lo
    hi_ = tk if hi is None else hi
    cols = []
    m128 = None
    for c in range(tk // CL):
        a, b = c * CL, (c + 1) * CL
        if b <= lo_ or a >= hi_:
            continue
        dc = d[:, a:b]
        if a < lo_ or b > hi_:
            cond = (lane[c] >= jnp.float32(lo_)) & (lane[c] < jnp.float32(hi_))
            dc = jnp.where(cond, dc, jnp.float32(jnp.inf))
        cols.append((c, dc))
    level = [dc for _, dc in cols]
    while len(level) > 1:
        nxt = [jnp.minimum(level[k], level[k + 1])
               for k in range(0, len(level) - 1, 2)]
        if len(level) % 2:
            nxt.append(level[-1])
        level = nxt
    m128 = level[0]
    lmin = jnp.min(m128, axis=1, keepdims=True)
    level = [jnp.where(dc == m128, lane[c], jnp.float32(3e7))
             for c, dc in cols]
    while len(level) > 1:
        nxt = [jnp.minimum(level[k], level[k + 1])
               for k in range(0, len(level) - 1, 2)]
        if len(level) % 2:
            nxt.append(level[-1])
        level = nxt
    kcand = jnp.where(m128 == lmin, level[0], jnp.float32(3e7))
    larg = (jnp.min(kcand, axis=1, keepdims=True)
            .astype(jnp.int32) + j * TK)
    pb = best_ref[pl.ds(w * TN, TN), :]
    take = lmin < pb
    barg_ref[pl.ds(w * TN, TN), :] = jnp.where(
        take, larg, barg_ref[pl.ds(w * TN, TN), :])
    best_ref[pl.ds(w * TN, TN), :] = jnp.where(take, lmin, pb)


def _argmin_body(x_ref, ct_ref, idx_ref, dmin_ref, best_ref, barg_ref):
    j = pl.program_id(1)
    x = x_ref[...]
    ct = ct_ref[...]
    mm = jnp.dot(x, ct, preferred_element_type=jnp.float32)
    sx = jnp.sum(x * x, axis=1, keepdims=True)
    sc = jnp.sum(ct * ct, axis=0, keepdims=True)
    d = (sx + sc) - 2.0 * mm
    lane0 = lax.broadcasted_iota(
        jnp.int32, (d.shape[0], CL), 1).astype(jnp.float32)
    lane = [lane0 + jnp.float32(c * CL) for c in range(TK // CL)]

    @pl.when(j == 0)
    def _():
        best_ref[...] = jnp.full((3 * TN, 1), jnp.inf, jnp.float32)
        barg_ref[...] = jnp.zeros((3 * TN, 1), jnp.int32)
        _upd(d, lane, j, None, None, best_ref, barg_ref, 0)

    @pl.when(j == 1)
    def _():
        _upd(d, lane, j, None, WB - TK, best_ref, barg_ref, 0)
        _upd(d, lane, j, WB - TK, None, best_ref, barg_ref, 1)

    @pl.when(j == 2)
    def _():
        _upd(d, lane, j, None, 2 * WB - 2 * TK, best_ref, barg_ref, 1)
        _upd(d, lane, j, 2 * WB - 2 * TK, None, best_ref, barg_ref, 2)

    @pl.when(j == NJ - 1)
    def _():
        _upd(d, lane, j, None, None, best_ref, barg_ref, 2)
        b0 = best_ref[pl.ds(0, TN), :]
        b1 = best_ref[pl.ds(TN, TN), :]
        b2 = best_ref[pl.ds(2 * TN, TN), :]
        i0 = barg_ref[pl.ds(0, TN), :]
        i1 = barg_ref[pl.ds(TN, TN), :]
        i2 = barg_ref[pl.ds(2 * TN, TN), :]
        v = b0.astype(jnp.bfloat16).astype(jnp.float32)
        i = i0
        e = b0
        t1 = b1 < v
        i = jnp.where(t1, i1, i)
        e = jnp.where(t1, b1, e)
        v = jnp.where(t1, b1.astype(jnp.bfloat16).astype(jnp.float32), v)
        t2 = b2 < v
        i = jnp.where(t2, i2, i)
        e = jnp.where(t2, b2, e)
        idx_ref[...] = i
        dmin_ref[...] = e


def _argmin_call(flat_x, ct):
    return pl.pallas_call(
        _argmin_body,
        grid=(NI, NJ),
        in_specs=[
            pl.BlockSpec((TN, DC), lambda i, j: (i, 0)),
            pl.BlockSpec((DC, TK), lambda i, j: (0, j)),
        ],
        out_specs=[
            pl.BlockSpec((TN, 1), lambda i, j: (i, 0)),
            pl.BlockSpec((TN, 1), lambda i, j: (i, 0)),
        ],
        out_shape=[
            jax.ShapeDtypeStruct((NT, 1), jnp.int32),
            jax.ShapeDtypeStruct((NT, 1), jnp.float32),
        ],
        scratch_shapes=[
            pltpu.VMEM((3 * TN, 1), jnp.float32),
            pltpu.VMEM((3 * TN, 1), jnp.int32),
        ],
        compiler_params=pltpu.CompilerParams(
            dimension_semantics=("parallel", "arbitrary"),
        ),
    )(flat_x, ct)


def _sc_gather(codebook, idx3):
    mesh = plsc.VectorSubcoreMesh(core_axis_name="c", subcore_axis_name="s")

    @functools.partial(
        pl.kernel,
        mesh=mesh,
        out_type=jax.ShapeDtypeStruct((NT, DC), jnp.float32),
        scratch_types=[
            pltpu.VMEM((CH,), jnp.int32),
            pltpu.VMEM((CH,), jnp.int32),
            pltpu.VMEM((CH,), jnp.int32),
            pltpu.VMEM((CH,), jnp.int32),
            pltpu.VMEM((CH, DC), jnp.float32),
            pltpu.SemaphoreType.DMA,
        ],
    )
    def kern(cb_hbm, idx_hbm, q_hbm,
             idx_v0, idx_v1, idx_v2, idx_v3, rows_v, sem):
        idx_vs = [idx_v0, idx_v1, idx_v2, idx_v3]
        cid = lax.axis_index("c")
        sid = lax.axis_index("s")
        wid = cid * 16 + sid
        base = wid * BPW

        for c in range(NCH):
            pltpu.sync_copy(idx_hbm.at[wid, c], idx_vs[c])
        for c in range(NCH):
            pltpu.async_copy(cb_hbm.at[idx_vs[c]], rows_v, sem).wait()
            pltpu.sync_copy(rows_v, q_hbm.at[pl.ds(base + c * CH, CH)])

    return kern(codebook, idx3)


TT = 512
KL = 1024
NG = KC // KL
NC_H = NT // TT


def _hist_body(idx_ref, cnt_ref, acc_ref):
    g = pl.program_id(0)
    t = pl.program_id(1)

    @pl.when(t == 0)
    def _():
        acc_ref[...] = jnp.zeros((1, KL), jnp.float32)

    idxv = idx_ref[...]
    bins = lax.broadcasted_iota(jnp.int32, (TT, KL), 1) + g * KL
    eq = (idxv == bins).astype(jnp.float32)
    acc_ref[...] += jnp.sum(eq, axis=0, keepdims=True)

    @pl.when(t == NC_H - 1)
    def _():
        cnt_ref[...] = acc_ref[...]


def _hist_call(idx):
    return pl.pallas_call(
        _hist_body,
        grid=(NG, NC_H),
        in_specs=[
            pl.BlockSpec((TT, 1), lambda g, t: (t, 0)),
        ],
        out_specs=pl.BlockSpec((1, KL), lambda g, t: (0, g)),
        out_shape=jax.ShapeDtypeStruct((1, KC), jnp.float32),
        scratch_shapes=[pltpu.VMEM((1, KL), jnp.float32)],
        compiler_params=pltpu.CompilerParams(
            dimension_semantics=("parallel", "arbitrary"),
        ),
    )(idx)


def _final_body(cnt_ref, dmin_ref, loss_ref, ppl_ref):
    cnt = cnt_ref[...]
    p = cnt * (1.0 / NT)
    ent = jnp.sum(p * jnp.log(p + 1e-10))
    ppl_ref[...] = jnp.exp(-ent).reshape(1, 1)
    loss_ref[...] = (jnp.sum(dmin_ref[...])
                     * (COMMITMENT_COST / (NT * DC))).reshape(1, 1)


def _final_call(cnt2, dmin2):
    return pl.pallas_call(
        _final_body,
        out_shape=[
            jax.ShapeDtypeStruct((1, 1), jnp.float32),
            jax.ShapeDtypeStruct((1, 1), jnp.float32),
        ],
    )(cnt2, dmin2)


def kernel(inputs, codebook):
    B, C, H, W = inputs.shape
    x = jnp.transpose(inputs, (0, 2, 3, 1))
    flat_x = x.reshape(NT, DC)
    ct = codebook.T

    idx, dmin = _argmin_call(flat_x, ct)

    idx3 = idx.reshape(NW, NCH, CH)
    q_flat = _sc_gather(codebook, idx3)

    cnt = _hist_call(idx)
    dmin2 = dmin.reshape(NI, TN)
    loss, ppl = _final_call(cnt, dmin2)

    quantized = q_flat.reshape(B, H, W, C)
    quantized_out = jnp.transpose(quantized, (0, 3, 1, 2))
    return (loss.reshape(()), quantized_out, ppl.reshape(()), idx)


# speedup vs baseline: 1.1096x; 1.1096x over previous
import functools

import jax
import jax.numpy as jnp
from jax import lax
from jax.experimental import pallas as pl
from jax.experimental.pallas import tpu as pltpu
from jax.experimental.pallas import tpu_sc as plsc

KC = 8192
DC = 256
NT = 16384
COMMITMENT_COST = 0.25

TN = 2048
TK = 2048
NI = NT // TN
NJ = KC // TK

WB = 2736
_BIG = 0x7FFFFFFF

NW = 32
BPW = NT // NW
CH = 128
NCH = BPW // CH


CL = 128


def _upd(d, lane, j, lo, hi, best_ref, barg_ref, w):
    tn, tk = d.shape
    lo_ = 0 if lo is None else lo
    hi_ = tk if hi is None else hi
    cols = []
    m128 = None
    for c in range(tk // CL):
        a, b = c * CL, (c + 1) * CL
        if b <= lo_ or a >= hi_:
            continue
        dc = d[:, a:b]
        if a < lo_ or b > hi_:
            cond = (lane[c] >= jnp.float32(lo_)) & (lane[c] < jnp.float32(hi_))
            dc = jnp.where(cond, dc, jnp.float32(jnp.inf))
        cols.append((c, dc))
    level = [dc for _, dc in cols]
    while len(level) > 1:
        nxt = [jnp.minimum(level[k], level[k + 1])
               for k in range(0, len(level) - 1, 2)]
        if len(level) % 2:
            nxt.append(level[-1])
        level = nxt
    m128 = level[0]
    lmin = jnp.min(m128, axis=1, keepdims=True)
    level = [jnp.where(dc == m128, lane[c], jnp.float32(3e7))
             for c, dc in cols]
    while len(level) > 1:
        nxt = [jnp.minimum(level[k], level[k + 1])
               for k in range(0, len(level) - 1, 2)]
        if len(level) % 2:
            nxt.append(level[-1])
        level = nxt
    kcand = jnp.where(m128 == lmin, level[0], jnp.float32(3e7))
    larg = (jnp.min(kcand, axis=1, keepdims=True)
            .astype(jnp.int32) + j * TK)
    pb = best_ref[pl.ds(w * TN, TN), :]
    take = lmin < pb
    barg_ref[pl.ds(w * TN, TN), :] = jnp.where(
        take, larg, barg_ref[pl.ds(w * TN, TN), :])
    best_ref[pl.ds(w * TN, TN), :] = jnp.where(take, lmin, pb)


def _argmin_body(x_ref, ct_ref, idx_ref, dmin_ref, best_ref, barg_ref):
    j = pl.program_id(1)
    x = x_ref[...]
    ct = ct_ref[...]
    mm = jnp.dot(x, ct, preferred_element_type=jnp.float32)
    sx = jnp.sum(x * x, axis=1, keepdims=True)
    sc = jnp.sum(ct * ct, axis=0, keepdims=True)
    d = (sx + sc) - 2.0 * mm
    lane0 = lax.broadcasted_iota(
        jnp.int32, (d.shape[0], CL), 1).astype(jnp.float32)
    lane = [lane0 + jnp.float32(c * CL) for c in range(TK // CL)]

    @pl.when(j == 0)
    def _():
        best_ref[...] = jnp.full((3 * TN, 1), jnp.inf, jnp.float32)
        barg_ref[...] = jnp.zeros((3 * TN, 1), jnp.int32)
        _upd(d, lane, j, None, None, best_ref, barg_ref, 0)

    @pl.when(j == 1)
    def _():
        _upd(d, lane, j, None, WB - TK, best_ref, barg_ref, 0)
        _upd(d, lane, j, WB - TK, None, best_ref, barg_ref, 1)

    @pl.when(j == 2)
    def _():
        _upd(d, lane, j, None, 2 * WB - 2 * TK, best_ref, barg_ref, 1)
        _upd(d, lane, j, 2 * WB - 2 * TK, None, best_ref, barg_ref, 2)

    @pl.when(j == NJ - 1)
    def _():
        _upd(d, lane, j, None, None, best_ref, barg_ref, 2)
        b0 = best_ref[pl.ds(0, TN), :]
        b1 = best_ref[pl.ds(TN, TN), :]
        b2 = best_ref[pl.ds(2 * TN, TN), :]
        i0 = barg_ref[pl.ds(0, TN), :]
        i1 = barg_ref[pl.ds(TN, TN), :]
        i2 = barg_ref[pl.ds(2 * TN, TN), :]
        v = b0.astype(jnp.bfloat16).astype(jnp.float32)
        i = i0
        e = b0
        t1 = b1 < v
        i = jnp.where(t1, i1, i)
        e = jnp.where(t1, b1, e)
        v = jnp.where(t1, b1.astype(jnp.bfloat16).astype(jnp.float32), v)
        t2 = b2 < v
        i = jnp.where(t2, i2, i)
        e = jnp.where(t2, b2, e)
        idx_ref[...] = i
        dmin_ref[...] = e


def _argmin_call(flat_x, ct):
    return pl.pallas_call(
        _argmin_body,
        grid=(NI, NJ),
        in_specs=[
            pl.BlockSpec((TN, DC), lambda i, j: (i, 0)),
            pl.BlockSpec((DC, TK), lambda i, j: (0, j)),
        ],
        out_specs=[
            pl.BlockSpec((TN, 1), lambda i, j: (i, 0)),
            pl.BlockSpec((TN, 1), lambda i, j: (i, 0)),
        ],
        out_shape=[
            jax.ShapeDtypeStruct((NT, 1), jnp.int32),
            jax.ShapeDtypeStruct((NT, 1), jnp.float32),
        ],
        scratch_shapes=[
            pltpu.VMEM((3 * TN, 1), jnp.float32),
            pltpu.VMEM((3 * TN, 1), jnp.int32),
        ],
        compiler_params=pltpu.CompilerParams(
            dimension_semantics=("parallel", "arbitrary"),
        ),
    )(flat_x, ct)


def _sc_gather(codebook, idx3):
    mesh = plsc.VectorSubcoreMesh(core_axis_name="c", subcore_axis_name="s")

    @functools.partial(
        pl.kernel,
        mesh=mesh,
        out_type=jax.ShapeDtypeStruct((NT, DC), jnp.float32),
        scratch_types=[
            pltpu.VMEM((CH,), jnp.int32),
            pltpu.VMEM((CH,), jnp.int32),
            pltpu.VMEM((CH,), jnp.int32),
            pltpu.VMEM((CH,), jnp.int32),
            pltpu.VMEM((CH, DC), jnp.float32),
            pltpu.SemaphoreType.DMA,
        ],
    )
    def kern(cb_hbm, idx_hbm, q_hbm,
             idx_v0, idx_v1, idx_v2, idx_v3, rows_v, sem):
        idx_vs = [idx_v0, idx_v1, idx_v2, idx_v3]
        cid = lax.axis_index("c")
        sid = lax.axis_index("s")
        wid = cid * 16 + sid
        base = wid * BPW

        for c in range(NCH):
            pltpu.sync_copy(idx_hbm.at[wid, c], idx_vs[c])
        for c in range(NCH):
            pltpu.async_copy(cb_hbm.at[idx_vs[c]], rows_v, sem).wait()
            pltpu.sync_copy(rows_v, q_hbm.at[pl.ds(base + c * CH, CH)])

    return kern(codebook, idx3)


TT = 512
KL = 1024
NG = KC // KL
NC_H = NT // TT


def _hist_body(idx_ref, cnt_ref, acc_ref):
    g = pl.program_id(0)
    t = pl.program_id(1)

    @pl.when(t == 0)
    def _():
        acc_ref[...] = jnp.zeros((1, KL), jnp.float32)

    idxv = idx_ref[...]
    bins = lax.broadcasted_iota(jnp.int32, (TT, KL), 1) + g * KL
    eq = (idxv == bins).astype(jnp.float32)
    acc_ref[...] += jnp.sum(eq, axis=0, keepdims=True)

    @pl.when(t == NC_H - 1)
    def _():
        cnt_ref[...] = acc_ref[...]


def _hist_call(idx):
    return pl.pallas_call(
        _hist_body,
        grid=(NG, NC_H),
        in_specs=[
            pl.BlockSpec((TT, 1), lambda g, t: (t, 0)),
        ],
        out_specs=pl.BlockSpec((1, KL), lambda g, t: (0, g)),
        out_shape=jax.ShapeDtypeStruct((1, KC), jnp.float32),
        scratch_shapes=[pltpu.VMEM((1, KL), jnp.float32)],
        compiler_params=pltpu.CompilerParams(
            dimension_semantics=("parallel", "arbitrary"),
        ),
    )(idx)


def _final_body(cnt_ref, dmin_ref, loss_ref, ppl_ref):
    cnt = cnt_ref[...]
    p = cnt * (1.0 / NT)
    ent = jnp.sum(p * jnp.log(p + 1e-10))
    ppl_ref[...] = jnp.exp(-ent).reshape(1, 1)
    loss_ref[...] = (jnp.sum(dmin_ref[...])
                     * (COMMITMENT_COST / (NT * DC))).reshape(1, 1)


def _final_call(cnt2, dmin2):
    return pl.pallas_call(
        _final_body,
        out_shape=[
            jax.ShapeDtypeStruct((1, 1), jnp.float32),
            jax.ShapeDtypeStruct((1, 1), jnp.float32),
        ],
    )(cnt2, dmin2)


def kernel(inputs, codebook):
    B, C, H, W = inputs.shape
    x = jnp.transpose(inputs, (0, 2, 3, 1))
    flat_x = x.reshape(NT, DC)
    ct = codebook.T

    idx, dmin = _argmin_call(flat_x, ct)

    idx3 = idx.reshape(NW, NCH, CH)
    q_flat = _sc_gather(codebook, idx3)

    cnt = _hist_call(idx)
    dmin2 = dmin.reshape(NI, TN)
    loss, ppl = _final_call(cnt, dmin2)

    quantized = q_flat.reshape(B, H, W, C)
    quantized_out = jnp.transpose(quantized, (0, 3, 1, 2))
    return (loss.reshape(()), quantized_out, ppl.reshape(()), idx)


# speedup vs baseline: 1.1099x; 1.0002x over previous
import functools

import jax
import jax.numpy as jnp
from jax import lax
from jax.experimental import pallas as pl
from jax.experimental.pallas import tpu as pltpu
from jax.experimental.pallas import tpu_sc as plsc

KC = 8192
DC = 256
NT = 16384
COMMITMENT_COST = 0.25

TN = 2048
TK = 2048
NI = NT // TN
NJ = KC // TK

WB = 2736

NW = 32
BPW = NT // NW
CH = 128
NCH = BPW // CH


CL = 128


def _upd(d, lane, j, lo, hi, best_ref, barg_ref, w):
    tn, tk = d.shape
    lo_ = 0 if lo is None else lo
    hi_ = tk if hi is None else hi
    cols = []
    m128 = None
    for c in range(tk // CL):
        a, b = c * CL, (c + 1) * CL
        if b <= lo_ or a >= hi_:
            continue
        dc = d[:, a:b]
        if a < lo_ or b > hi_:
            cond = (lane[c] >= jnp.float32(lo_)) & (lane[c] < jnp.float32(hi_))
            dc = jnp.where(cond, dc, jnp.float32(jnp.inf))
        cols.append((c, dc))
    level = [dc for _, dc in cols]
    while len(level) > 1:
        nxt = [jnp.minimum(level[k], level[k + 1])
               for k in range(0, len(level) - 1, 2)]
        if len(level) % 2:
            nxt.append(level[-1])
        level = nxt
    m128 = level[0]
    lmin = jnp.min(m128, axis=1, keepdims=True)
    level = [jnp.where(dc == m128, lane[c], jnp.float32(3e7))
             for c, dc in cols]
    while len(level) > 1:
        nxt = [jnp.minimum(level[k], level[k + 1])
               for k in range(0, len(level) - 1, 2)]
        if len(level) % 2:
            nxt.append(level[-1])
        level = nxt
    kcand = jnp.where(m128 == lmin, level[0], jnp.float32(3e7))
    larg = (jnp.min(kcand, axis=1, keepdims=True)
            .astype(jnp.int32) + j * TK)
    pb = best_ref[pl.ds(w * TN, TN), :]
    take = lmin < pb
    barg_ref[pl.ds(w * TN, TN), :] = jnp.where(
        take, larg, barg_ref[pl.ds(w * TN, TN), :])
    best_ref[pl.ds(w * TN, TN), :] = jnp.where(take, lmin, pb)


def _argmin_body(x_ref, ct_ref, idx_ref, dmin_ref, best_ref, barg_ref):
    j = pl.program_id(1)
    x = x_ref[...]
    ct = ct_ref[...]
    mm = jnp.dot(x, ct, preferred_element_type=jnp.float32)
    sx = jnp.sum(x * x, axis=1, keepdims=True)
    sc = jnp.sum(ct * ct, axis=0, keepdims=True)
    d = (sx + sc) - 2.0 * mm
    lane0 = lax.broadcasted_iota(
        jnp.int32, (d.shape[0], CL), 1).astype(jnp.float32)
    lane = [lane0 + jnp.float32(c * CL) for c in range(TK // CL)]

    @pl.when(j == 0)
    def _():
        best_ref[...] = jnp.full((3 * TN, 1), jnp.inf, jnp.float32)
        barg_ref[...] = jnp.zeros((3 * TN, 1), jnp.int32)
        _upd(d, lane, j, None, None, best_ref, barg_ref, 0)

    @pl.when(j == 1)
    def _():
        _upd(d, lane, j, None, WB - TK, best_ref, barg_ref, 0)
        _upd(d, lane, j, WB - TK, None, best_ref, barg_ref, 1)

    @pl.when(j == 2)
    def _():
        _upd(d, lane, j, None, 2 * WB - 2 * TK, best_ref, barg_ref, 1)
        _upd(d, lane, j, 2 * WB - 2 * TK, None, best_ref, barg_ref, 2)

    @pl.when(j == NJ - 1)
    def _():
        _upd(d, lane, j, None, None, best_ref, barg_ref, 2)
        b0 = best_ref[pl.ds(0, TN), :]
        b1 = best_ref[pl.ds(TN, TN), :]
        b2 = best_ref[pl.ds(2 * TN, TN), :]
        i0 = barg_ref[pl.ds(0, TN), :]
        i1 = barg_ref[pl.ds(TN, TN), :]
        i2 = barg_ref[pl.ds(2 * TN, TN), :]
        v = b0.astype(jnp.bfloat16).astype(jnp.float32)
        i = i0
        e = b0
        t1 = b1 < v
        i = jnp.where(t1, i1, i)
        e = jnp.where(t1, b1, e)
        v = jnp.where(t1, b1.astype(jnp.bfloat16).astype(jnp.float32), v)
        t2 = b2 < v
        i = jnp.where(t2, i2, i)
        e = jnp.where(t2, b2, e)
        idx_ref[...] = i
        dmin_ref[...] = e


def _argmin_call(flat_x, ct):
    return pl.pallas_call(
        _argmin_body,
        grid=(NI, NJ),
        in_specs=[
            pl.BlockSpec((TN, DC), lambda i, j: (i, 0)),
            pl.BlockSpec((DC, TK), lambda i, j: (0, j)),
        ],
        out_specs=[
            pl.BlockSpec((TN, 1), lambda i, j: (i, 0)),
            pl.BlockSpec((TN, 1), lambda i, j: (i, 0)),
        ],
        out_shape=[
            jax.ShapeDtypeStruct((NT, 1), jnp.int32),
            jax.ShapeDtypeStruct((NT, 1), jnp.float32),
        ],
        scratch_shapes=[
            pltpu.VMEM((3 * TN, 1), jnp.float32),
            pltpu.VMEM((3 * TN, 1), jnp.int32),
        ],
        compiler_params=pltpu.CompilerParams(
            dimension_semantics=("parallel", "arbitrary"),
        ),
    )(flat_x, ct)


def _sc_gather(codebook, idx3):
    mesh = plsc.VectorSubcoreMesh(core_axis_name="c", subcore_axis_name="s")

    @functools.partial(
        pl.kernel,
        mesh=mesh,
        out_type=jax.ShapeDtypeStruct((NT, DC), jnp.float32),
        scratch_types=[
            pltpu.VMEM((CH,), jnp.int32),
            pltpu.VMEM((CH,), jnp.int32),
            pltpu.VMEM((CH,), jnp.int32),
            pltpu.VMEM((CH,), jnp.int32),
            pltpu.VMEM((CH, DC), jnp.float32),
            pltpu.SemaphoreType.DMA,
        ],
    )
    def kern(cb_hbm, idx_hbm, q_hbm,
             idx_v0, idx_v1, idx_v2, idx_v3, rows_v, sem):
        idx_vs = [idx_v0, idx_v1, idx_v2, idx_v3]
        cid = lax.axis_index("c")
        sid = lax.axis_index("s")
        wid = cid * 16 + sid
        base = wid * BPW

        for c in range(NCH):
            pltpu.sync_copy(idx_hbm.at[wid, c], idx_vs[c])
        for c in range(NCH):
            pltpu.async_copy(cb_hbm.at[idx_vs[c]], rows_v, sem).wait()
            pltpu.sync_copy(rows_v, q_hbm.at[pl.ds(base + c * CH, CH)])

    return kern(codebook, idx3)


TT = 512
KL = 1024
NG = KC // KL
NC_H = NT // TT


def _hist_body(idx_ref, cnt_ref, acc_ref):
    g = pl.program_id(0)
    t = pl.program_id(1)

    @pl.when(t == 0)
    def _():
        acc_ref[...] = jnp.zeros((1, KL), jnp.float32)

    idxv = idx_ref[...]
    bins = lax.broadcasted_iota(jnp.int32, (TT, KL), 1) + g * KL
    eq = (idxv == bins).astype(jnp.float32)
    acc_ref[...] += jnp.sum(eq, axis=0, keepdims=True)

    @pl.when(t == NC_H - 1)
    def _():
        cnt_ref[...] = acc_ref[...]


def _hist_call(idx):
    return pl.pallas_call(
        _hist_body,
        grid=(NG, NC_H),
        in_specs=[
            pl.BlockSpec((TT, 1), lambda g, t: (t, 0)),
        ],
        out_specs=pl.BlockSpec((1, KL), lambda g, t: (0, g)),
        out_shape=jax.ShapeDtypeStruct((1, KC), jnp.float32),
        scratch_shapes=[pltpu.VMEM((1, KL), jnp.float32)],
        compiler_params=pltpu.CompilerParams(
            dimension_semantics=("parallel", "arbitrary"),
        ),
    )(idx)


def _final_body(cnt_ref, dmin_ref, loss_ref, ppl_ref):
    cnt = cnt_ref[...]
    p = cnt * (1.0 / NT)
    ent = jnp.sum(p * jnp.log(p + 1e-10))
    ppl_ref[...] = jnp.exp(-ent).reshape(1, 1)
    loss_ref[...] = (jnp.sum(dmin_ref[...])
                     * (COMMITMENT_COST / (NT * DC))).reshape(1, 1)


def _final_call(cnt2, dmin2):
    return pl.pallas_call(
        _final_body,
        out_shape=[
            jax.ShapeDtypeStruct((1, 1), jnp.float32),
            jax.ShapeDtypeStruct((1, 1), jnp.float32),
        ],
    )(cnt2, dmin2)


def kernel(inputs, codebook):
    B, C, H, W = inputs.shape
    x = jnp.transpose(inputs, (0, 2, 3, 1))
    flat_x = x.reshape(NT, DC)
    ct = codebook.T

    idx, dmin = _argmin_call(flat_x, ct)

    idx3 = idx.reshape(NW, NCH, CH)
    q_flat = _sc_gather(codebook, idx3)

    cnt = _hist_call(idx)
    dmin2 = dmin.reshape(NI, TN)
    loss, ppl = _final_call(cnt, dmin2)

    quantized = q_flat.reshape(B, H, W, C)
    quantized_out = jnp.transpose(quantized, (0, 3, 1, 2))
    return (loss.reshape(()), quantized_out, ppl.reshape(()), idx)
